# baseline (speedup 1.0000x reference)
import functools

import jax
import jax.numpy as jnp
from jax import lax
from jax.experimental import pallas as pl
from jax.experimental.pallas import tpu as pltpu
from jax.experimental.pallas import tpu_sc as plsc

N = 10000
NP = 10240
D = 128
E = 320000
CHUNK = 128
NCHUNKS = E // CHUNK
NC, NS = 2, 16
NW = NC * NS
ROWS_PER_TILE = NP // NS
ZROWS = CHUNK

_sc_mesh = plsc.VectorSubcoreMesh(
    core_axis_name="c", subcore_axis_name="s", num_cores=NC, num_subcores=NS)


NCHUNKSP = 2560
CPW = NCHUNKSP // NW
DGRP = 8


def _deg_body(dst2_hbm, ones_hbm, zeros_hbm, out_hbm, acc_sh, didx_v, ones_v,
              zbuf_v, *dsems):
    cid = lax.axis_index("c")
    sid = lax.axis_index("s")
    wid = sid * NC + cid
    base = sid * ROWS_PER_TILE
    pltpu.sync_copy(zeros_hbm, zbuf_v)
    pltpu.sync_copy(zbuf_v, acc_sh.at[pl.ds(base, ROWS_PER_TILE)])
    pltpu.sync_copy(ones_hbm, ones_v)
    pltpu.sync_copy(dst2_hbm.at[pl.ds(wid * CPW, CPW)], didx_v)
    plsc.subcore_barrier()

    def body(g, carry):
        c0 = g * DGRP
        descs = [
            pltpu.async_copy(ones_v, acc_sh.at[didx_v.at[c0 + b]], dsems[b],
                             add=True)
            for b in range(DGRP)
        ]
        for d in descs:
            d.wait()
        return carry

    lax.fori_loop(0, CPW // DGRP, body, 0)
    plsc.subcore_barrier()
    pltpu.sync_copy(acc_sh.at[pl.ds(base, ROWS_PER_TILE)],
                    out_hbm.at[pl.ds(cid * NP + base, ROWS_PER_TILE)])


_deg_call = pl.kernel(
    _deg_body,
    out_type=jax.ShapeDtypeStruct((NC * NP,), jnp.float32),
    mesh=_sc_mesh,
    scratch_types=[
        pltpu.VMEM_SHARED((NP,), jnp.float32),
        pltpu.VMEM((CPW, CHUNK), jnp.int32),
        pltpu.VMEM((CHUNK,), jnp.float32),
        pltpu.VMEM((ROWS_PER_TILE,), jnp.float32),
    ] + [pltpu.SemaphoreType.DMA] * DGRP,
)


NB = 2
NH = 2
HC = CPW // NH
NGRP = HC // NB


def _scatter_body(g_hbm, src2_hbm, dst2_hbm, zeros_hbm, out_hbm, acc_sh,
                  sidx_v, didx_v, rows_v, *sems):
    gsems, ssems = sems[:NB], sems[NB:]
    cid = lax.axis_index("c")
    sid = lax.axis_index("s")
    wid = sid * NC + cid
    base = sid * ROWS_PER_TILE
    start = wid * CPW
    def _fire(c, b):
        cc = jnp.minimum(c, HC - 1)
        pltpu.async_copy(g_hbm.at[sidx_v.at[cc]], rows_v.at[b], gsems[b])

    def _gwait(b):
        pltpu.make_async_copy(g_hbm.at[sidx_v.at[0]], rows_v.at[b],
                              gsems[b]).wait()

    def _scat(c, b):
        pltpu.sync_copy(rows_v.at[b], acc_sh.at[didx_v.at[c]], add=True)

    zdescs = [
        pltpu.async_copy(zeros_hbm,
                         acc_sh.at[pl.ds(base + k * ZROWS, ZROWS)],
                         ssems[k % NB])
        for k in range(ROWS_PER_TILE // ZROWS)
    ]

    for h in range(NH):
        pltpu.sync_copy(src2_hbm.at[pl.ds(start + h * HC, HC)], sidx_v)
        pltpu.sync_copy(dst2_hbm.at[pl.ds(start + h * HC, HC)], didx_v)
        _fire(0, 0)
        if h == 0:
            for d in zdescs:
                d.wait()
            plsc.subcore_barrier()

        def body(p, carry):
            c0 = 2 * p
            _fire(c0 + 1, 1)
            _gwait(0)
            _scat(c0, 0)
            _fire(c0 + 2, 0)
            _gwait(1)
            _scat(c0 + 1, 1)
            return carry

        lax.fori_loop(0, HC // 2, body, 0)
        _gwait(0)

    plsc.subcore_barrier()
    pltpu.sync_copy(acc_sh.at[pl.ds(base, ROWS_PER_TILE)],
                    out_hbm.at[cid, pl.ds(base, ROWS_PER_TILE)])


_scatter_call = pl.kernel(
    _scatter_body,
    out_type=jax.ShapeDtypeStruct((NC, NP, D), jnp.float32),
    mesh=_sc_mesh,
    scratch_types=[
        pltpu.VMEM_SHARED((NP, D), jnp.float32),
        pltpu.VMEM((HC, CHUNK), jnp.int32),
        pltpu.VMEM((HC, CHUNK), jnp.int32),
        pltpu.VMEM((NB, CHUNK, D), jnp.float32),
    ] + [pltpu.SemaphoreType.DMA] * (2 * NB),
)

BN = 1000
GRID = N // BN


def _tc_first_body(degp_ref, x_ref, w_ref, g_ref, dinv_ref):
    dp = degp_ref[...]
    deg = dp[0, :, 0:1] + dp[1, :, 0:1] + 1.0
    dinv = lax.rsqrt(deg)
    h = jnp.dot(x_ref[...], w_ref[...], preferred_element_type=jnp.float32)
    g_ref[...] = dinv * h
    dinv_ref[...] = dinv


_tc_first = pl.pallas_call(
    _tc_first_body,
    grid=(GRID,),
    in_specs=[
        pl.BlockSpec((NC, BN, 1), lambda i: (0, i, 0)),
        pl.BlockSpec((BN, D), lambda i: (i, 0)),
        pl.BlockSpec((D, D), lambda i: (0, 0)),
    ],
    out_specs=[
        pl.BlockSpec((BN, D), lambda i: (i, 0)),
        pl.BlockSpec((BN, 1), lambda i: (i, 0)),
    ],
    out_shape=[
        jax.ShapeDtypeStruct((N, D), jnp.float32),
        jax.ShapeDtypeStruct((N, 1), jnp.float32),
    ],
)


def _tc_mid_body(sp_ref, g_ref, dinv_ref, b_ref, w_ref, gout_ref):
    s = sp_ref[0] + sp_ref[1]
    dinv = dinv_ref[...]
    t = dinv * (s + g_ref[...]) + b_ref[...]
    xl = jnp.maximum(t, 0.0)
    h = jnp.dot(xl, w_ref[...], preferred_element_type=jnp.float32)
    gout_ref[...] = dinv * h


_tc_mid = pl.pallas_call(
    _tc_mid_body,
    grid=(GRID,),
    in_specs=[
        pl.BlockSpec((NC, BN, D), lambda i: (0, i, 0)),
        pl.BlockSpec((BN, D), lambda i: (i, 0)),
        pl.BlockSpec((BN, 1), lambda i: (i, 0)),
        pl.BlockSpec((1, D), lambda i: (0, 0)),
        pl.BlockSpec((D, D), lambda i: (0, 0)),
    ],
    out_specs=pl.BlockSpec((BN, D), lambda i: (i, 0)),
    out_shape=jax.ShapeDtypeStruct((N, D), jnp.float32),
)


def _tc_final_body(sp_ref, g_ref, dinv_ref, b_ref, out_ref):
    s = sp_ref[0] + sp_ref[1]
    out_ref[...] = dinv_ref[...] * (s + g_ref[...]) + b_ref[...]


_tc_final = pl.pallas_call(
    _tc_final_body,
    grid=(GRID,),
    in_specs=[
        pl.BlockSpec((NC, BN, D), lambda i: (0, i, 0)),
        pl.BlockSpec((BN, D), lambda i: (i, 0)),
        pl.BlockSpec((BN, 1), lambda i: (i, 0)),
        pl.BlockSpec((1, D), lambda i: (0, 0)),
    ],
    out_specs=pl.BlockSpec((BN, D), lambda i: (i, 0)),
    out_shape=jax.ShapeDtypeStruct((N, D), jnp.float32),
)


def kernel(x, edge_index, W1, b1, W2, b2, W3, b3):
    src = edge_index[0].astype(jnp.int32)
    dst = edge_index[1].astype(jnp.int32)
    zeros_l = jnp.zeros((ZROWS, D), jnp.float32)
    zeros_d = jnp.zeros((ROWS_PER_TILE,), jnp.float32)
    ones_d = jnp.ones((CHUNK,), jnp.float32)

    npad = (NCHUNKSP - NCHUNKS) * CHUNK
    pads = jnp.arange(npad, dtype=jnp.int32) % N
    src2 = jnp.concatenate([src, pads]).reshape(NCHUNKSP, CHUNK)
    padv = N + jnp.arange(npad, dtype=jnp.int32) % (NP - N)
    dst2 = jnp.concatenate([dst, padv]).reshape(NCHUNKSP, CHUNK)
    degp = _deg_call(dst2, ones_d, zeros_d).reshape(NC, NP, 1)
    g1, dinv = _tc_first(degp, x, W1)
    s1 = _scatter_call(g1, src2, dst2, zeros_l)
    g2 = _tc_mid(s1, g1, dinv, b1.reshape(1, D), W2)
    s2 = _scatter_call(g2, src2, dst2, zeros_l)
    g3 = _tc_mid(s2, g2, dinv, b2.reshape(1, D), W3)
    s3 = _scatter_call(g3, src2, dst2, zeros_l)
    return _tc_final(s3, g3, dinv, b3.reshape(1, D))


# speedup vs baseline: 1.0517x; 1.0517x over previous
import functools

import jax
import jax.numpy as jnp
from jax import lax
from jax.experimental import pallas as pl
from jax.experimental.pallas import tpu as pltpu
from jax.experimental.pallas import tpu_sc as plsc

N = 10000
NP = 10240
D = 128
E = 320000
CHUNK = 128
NCHUNKS = E // CHUNK
NC, NS = 2, 16
NW = NC * NS
ROWS_PER_TILE = NP // NS
ZROWS = CHUNK

_sc_mesh = plsc.VectorSubcoreMesh(
    core_axis_name="c", subcore_axis_name="s", num_cores=NC, num_subcores=NS)


NCHUNKSP = 2560
CPW = NCHUNKSP // NW
DGRP = 8


def _deg_body(dst2_hbm, ones_hbm, zeros_hbm, out_hbm, acc_sh, didx_v, ones_v,
              zbuf_v, *dsems):
    cid = lax.axis_index("c")
    sid = lax.axis_index("s")
    wid = sid * NC + cid
    base = sid * ROWS_PER_TILE
    pltpu.sync_copy(zeros_hbm, zbuf_v)
    pltpu.sync_copy(zbuf_v, acc_sh.at[pl.ds(base, ROWS_PER_TILE)])
    pltpu.sync_copy(ones_hbm, ones_v)
    pltpu.sync_copy(dst2_hbm.at[pl.ds(wid * CPW, CPW)], didx_v)
    plsc.subcore_barrier()

    def body(g, carry):
        c0 = g * DGRP
        descs = [
            pltpu.async_copy(ones_v, acc_sh.at[didx_v.at[c0 + b]], dsems[b],
                             add=True)
            for b in range(DGRP)
        ]
        for d in descs:
            d.wait()
        return carry

    lax.fori_loop(0, CPW // DGRP, body, 0)
    plsc.subcore_barrier()
    pltpu.sync_copy(acc_sh.at[pl.ds(base, ROWS_PER_TILE)],
                    out_hbm.at[pl.ds(cid * NP + base, ROWS_PER_TILE)])


_deg_call = pl.kernel(
    _deg_body,
    out_type=jax.ShapeDtypeStruct((NC * NP,), jnp.float32),
    mesh=_sc_mesh,
    scratch_types=[
        pltpu.VMEM_SHARED((NP,), jnp.float32),
        pltpu.VMEM((CPW, CHUNK), jnp.int32),
        pltpu.VMEM((CHUNK,), jnp.float32),
        pltpu.VMEM((ROWS_PER_TILE,), jnp.float32),
    ] + [pltpu.SemaphoreType.DMA] * DGRP,
)


NB = 2
NH = 2
HC = CPW // NH
NGRP = HC // NB


def _scatter_body(g_hbm, src2_hbm, dst2_hbm, zeros_hbm, out_hbm, acc_sh,
                  sidx_v, didx_v, rows_v, *sems):
    gsems, ssems = sems[:NB], sems[NB:]
    cid = lax.axis_index("c")
    sid = lax.axis_index("s")
    wid = sid * NC + cid
    base = sid * ROWS_PER_TILE
    start = wid * CPW
    pltpu.sync_copy(zeros_hbm, rows_v.at[0])
    zdescs = [
        pltpu.async_copy(rows_v.at[0], acc_sh.at[pl.ds(base + k * ZROWS,
                                                       ZROWS)],
                         ssems[k % NB])
        for k in range(ROWS_PER_TILE // ZROWS)
    ]
    for d in zdescs:
        d.wait()
    plsc.subcore_barrier()

    def _fire(c, b):
        cc = jnp.minimum(c, HC - 1)
        pltpu.async_copy(g_hbm.at[sidx_v.at[cc]], rows_v.at[b], gsems[b])

    def _gwait(b):
        pltpu.make_async_copy(g_hbm.at[sidx_v.at[0]], rows_v.at[b],
                              gsems[b]).wait()

    def _scat(c, b):
        pltpu.sync_copy(rows_v.at[b], acc_sh.at[didx_v.at[c]], add=True)

    for h in range(NH):
        pltpu.sync_copy(src2_hbm.at[pl.ds(start + h * HC, HC)], sidx_v)
        pltpu.sync_copy(dst2_hbm.at[pl.ds(start + h * HC, HC)], didx_v)
        _fire(0, 0)

        def body(p, carry):
            c0 = 2 * p
            _fire(c0 + 1, 1)
            _gwait(0)
            _scat(c0, 0)
            _fire(c0 + 2, 0)
            _gwait(1)
            _scat(c0 + 1, 1)
            return carry

        lax.fori_loop(0, HC // 2, body, 0)
        _gwait(0)

    plsc.subcore_barrier()
    pltpu.sync_copy(acc_sh.at[pl.ds(base, ROWS_PER_TILE)],
                    out_hbm.at[cid, pl.ds(base, ROWS_PER_TILE)])


_scatter_call = pl.kernel(
    _scatter_body,
    out_type=jax.ShapeDtypeStruct((NC, NP, D), jnp.float32),
    mesh=_sc_mesh,
    scratch_types=[
        pltpu.VMEM_SHARED((NP, D), jnp.float32),
        pltpu.VMEM((HC, CHUNK), jnp.int32),
        pltpu.VMEM((HC, CHUNK), jnp.int32),
        pltpu.VMEM((NB, CHUNK, D), jnp.float32),
    ] + [pltpu.SemaphoreType.DMA] * (2 * NB),
)

BN = 2000
GRID = N // BN


def _tc_first_body(degp_ref, x_ref, w_ref, g_ref, dinv_ref):
    dp = degp_ref[...]
    deg = dp[0, :, 0:1] + dp[1, :, 0:1] + 1.0
    dinv = lax.rsqrt(deg)
    h = jnp.dot(x_ref[...], w_ref[...], preferred_element_type=jnp.float32)
    g_ref[...] = dinv * h
    dinv_ref[...] = dinv


_tc_first = pl.pallas_call(
    _tc_first_body,
    grid=(GRID,),
    in_specs=[
        pl.BlockSpec((NC, BN, 1), lambda i: (0, i, 0)),
        pl.BlockSpec((BN, D), lambda i: (i, 0)),
        pl.BlockSpec((D, D), lambda i: (0, 0)),
    ],
    out_specs=[
        pl.BlockSpec((BN, D), lambda i: (i, 0)),
        pl.BlockSpec((BN, 1), lambda i: (i, 0)),
    ],
    out_shape=[
        jax.ShapeDtypeStruct((N, D), jnp.float32),
        jax.ShapeDtypeStruct((N, 1), jnp.float32),
    ],
)


def _tc_mid_body(sp_ref, g_ref, dinv_ref, b_ref, w_ref, gout_ref):
    s = sp_ref[0] + sp_ref[1]
    dinv = dinv_ref[...]
    t = dinv * (s + g_ref[...]) + b_ref[...]
    xl = jnp.maximum(t, 0.0)
    h = jnp.dot(xl, w_ref[...], preferred_element_type=jnp.float32)
    gout_ref[...] = dinv * h


_tc_mid = pl.pallas_call(
    _tc_mid_body,
    grid=(GRID,),
    in_specs=[
        pl.BlockSpec((NC, BN, D), lambda i: (0, i, 0)),
        pl.BlockSpec((BN, D), lambda i: (i, 0)),
        pl.BlockSpec((BN, 1), lambda i: (i, 0)),
        pl.BlockSpec((1, D), lambda i: (0, 0)),
        pl.BlockSpec((D, D), lambda i: (0, 0)),
    ],
    out_specs=pl.BlockSpec((BN, D), lambda i: (i, 0)),
    out_shape=jax.ShapeDtypeStruct((N, D), jnp.float32),
)


def _tc_final_body(sp_ref, g_ref, dinv_ref, b_ref, out_ref):
    s = sp_ref[0] + sp_ref[1]
    out_ref[...] = dinv_ref[...] * (s + g_ref[...]) + b_ref[...]


_tc_final = pl.pallas_call(
    _tc_final_body,
    grid=(GRID,),
    in_specs=[
        pl.BlockSpec((NC, BN, D), lambda i: (0, i, 0)),
        pl.BlockSpec((BN, D), lambda i: (i, 0)),
        pl.BlockSpec((BN, 1), lambda i: (i, 0)),
        pl.BlockSpec((1, D), lambda i: (0, 0)),
    ],
    out_specs=pl.BlockSpec((BN, D), lambda i: (i, 0)),
    out_shape=jax.ShapeDtypeStruct((N, D), jnp.float32),
)


def kernel(x, edge_index, W1, b1, W2, b2, W3, b3):
    src = edge_index[0].astype(jnp.int32)
    dst = edge_index[1].astype(jnp.int32)
    zeros_l = jnp.zeros((ZROWS, D), jnp.float32)
    zeros_d = jnp.zeros((ROWS_PER_TILE,), jnp.float32)
    ones_d = jnp.ones((CHUNK,), jnp.float32)

    npad = (NCHUNKSP - NCHUNKS) * CHUNK
    pads = jnp.arange(npad, dtype=jnp.int32) % N
    src2 = jnp.concatenate([src, pads]).reshape(NCHUNKSP, CHUNK)
    padv = N + jnp.arange(npad, dtype=jnp.int32) % (NP - N)
    dst2 = jnp.concatenate([dst, padv]).reshape(NCHUNKSP, CHUNK)
    degp = _deg_call(dst2, ones_d, zeros_d).reshape(NC, NP, 1)
    g1, dinv = _tc_first(degp, x, W1)
    s1 = _scatter_call(g1, src2, dst2, zeros_l)
    g2 = _tc_mid(s1, g1, dinv, b1.reshape(1, D), W2)
    s2 = _scatter_call(g2, src2, dst2, zeros_l)
    g3 = _tc_mid(s2, g2, dinv, b2.reshape(1, D), W3)
    s3 = _scatter_call(g3, src2, dst2, zeros_l)
    return _tc_final(s3, g3, dinv, b3.reshape(1, D))


# speedup vs baseline: 1.0533x; 1.0015x over previous
import jax
import jax.numpy as jnp
from jax import lax
from jax.experimental import pallas as pl
from jax.experimental.pallas import tpu as pltpu
from jax.experimental.pallas import tpu_sc as plsc

N = 10000
NP = 10240
D = 128
E = 320000
CHUNK = 128
NCHUNKS = E // CHUNK
NC, NS = 2, 16
NW = NC * NS
ROWS_PER_TILE = NP // NS
ZROWS = CHUNK

_sc_mesh = plsc.VectorSubcoreMesh(
    core_axis_name="c", subcore_axis_name="s", num_cores=NC, num_subcores=NS)


NCHUNKSP = 2560
CPW = NCHUNKSP // NW
DGRP = 8


def _deg_body(dst2_hbm, ones_hbm, zeros_hbm, out_hbm, acc_sh, didx_v, ones_v,
              zbuf_v, *dsems):
    cid = lax.axis_index("c")
    sid = lax.axis_index("s")
    wid = sid * NC + cid
    base = sid * ROWS_PER_TILE
    pltpu.sync_copy(zeros_hbm, zbuf_v)
    pltpu.sync_copy(zbuf_v, acc_sh.at[pl.ds(base, ROWS_PER_TILE)])
    pltpu.sync_copy(ones_hbm, ones_v)
    pltpu.sync_copy(dst2_hbm.at[pl.ds(wid * CPW, CPW)], didx_v)
    plsc.subcore_barrier()

    def body(g, carry):
        c0 = g * DGRP
        descs = [
            pltpu.async_copy(ones_v, acc_sh.at[didx_v.at[c0 + b]], dsems[b],
                             add=True)
            for b in range(DGRP)
        ]
        for d in descs:
            d.wait()
        return carry

    lax.fori_loop(0, CPW // DGRP, body, 0)
    plsc.subcore_barrier()
    pltpu.sync_copy(acc_sh.at[pl.ds(base, ROWS_PER_TILE)],
                    out_hbm.at[pl.ds(cid * NP + base, ROWS_PER_TILE)])


_deg_call = pl.kernel(
    _deg_body,
    out_type=jax.ShapeDtypeStruct((NC * NP,), jnp.float32),
    mesh=_sc_mesh,
    scratch_types=[
        pltpu.VMEM_SHARED((NP,), jnp.float32),
        pltpu.VMEM((CPW, CHUNK), jnp.int32),
        pltpu.VMEM((CHUNK,), jnp.float32),
        pltpu.VMEM((ROWS_PER_TILE,), jnp.float32),
    ] + [pltpu.SemaphoreType.DMA] * DGRP,
)


NB = 2
NH = 2
HC = CPW // NH
NGRP = HC // NB


def _scatter_body(g_hbm, src2_hbm, dst2_hbm, zeros_hbm, out_hbm, acc_sh,
                  sidx_v, didx_v, rows_v, *sems):
    gsems, ssems = sems[:NB], sems[NB:]
    cid = lax.axis_index("c")
    sid = lax.axis_index("s")
    wid = sid * NC + cid
    base = sid * ROWS_PER_TILE
    start = wid * CPW
    pltpu.sync_copy(zeros_hbm, rows_v.at[0])
    zdescs = [
        pltpu.async_copy(rows_v.at[0], acc_sh.at[pl.ds(base + k * ZROWS,
                                                       ZROWS)],
                         ssems[k % NB])
        for k in range(ROWS_PER_TILE // ZROWS)
    ]
    for d in zdescs:
        d.wait()
    plsc.subcore_barrier()

    def _fire(c, b):
        cc = jnp.minimum(c, HC - 1)
        pltpu.async_copy(g_hbm.at[sidx_v.at[cc]], rows_v.at[b], gsems[b])

    def _gwait(b):
        pltpu.make_async_copy(g_hbm.at[sidx_v.at[0]], rows_v.at[b],
                              gsems[b]).wait()

    def _scat(c, b):
        pltpu.sync_copy(rows_v.at[b], acc_sh.at[didx_v.at[c]], add=True)

    for h in range(NH):
        pltpu.sync_copy(src2_hbm.at[pl.ds(start + h * HC, HC)], sidx_v)
        pltpu.sync_copy(dst2_hbm.at[pl.ds(start + h * HC, HC)], didx_v)
        _fire(0, 0)

        def body(p, carry):
            c0 = 2 * p
            _fire(c0 + 1, 1)
            _gwait(0)
            _scat(c0, 0)
            _fire(c0 + 2, 0)
            _gwait(1)
            _scat(c0 + 1, 1)
            return carry

        lax.fori_loop(0, HC // 2, body, 0)
        _gwait(0)

    plsc.subcore_barrier()
    pltpu.sync_copy(acc_sh.at[pl.ds(base, ROWS_PER_TILE)],
                    out_hbm.at[cid, pl.ds(base, ROWS_PER_TILE)])


_scatter_call = pl.kernel(
    _scatter_body,
    out_type=jax.ShapeDtypeStruct((NC, NP, D), jnp.float32),
    mesh=_sc_mesh,
    scratch_types=[
        pltpu.VMEM_SHARED((NP, D), jnp.float32),
        pltpu.VMEM((HC, CHUNK), jnp.int32),
        pltpu.VMEM((HC, CHUNK), jnp.int32),
        pltpu.VMEM((NB, CHUNK, D), jnp.float32),
    ] + [pltpu.SemaphoreType.DMA] * (2 * NB),
)

BN = 2000
GRID = N // BN


def _tc_first_body(degp_ref, x_ref, w_ref, g_ref, dinv_ref):
    dp = degp_ref[...]
    deg = dp[0, :, 0:1] + dp[1, :, 0:1] + 1.0
    dinv = lax.rsqrt(deg)
    h = jnp.dot(x_ref[...], w_ref[...], preferred_element_type=jnp.float32)
    g_ref[...] = dinv * h
    dinv_ref[...] = dinv


_tc_first = pl.pallas_call(
    _tc_first_body,
    grid=(GRID,),
    in_specs=[
        pl.BlockSpec((NC, BN, 1), lambda i: (0, i, 0)),
        pl.BlockSpec((BN, D), lambda i: (i, 0)),
        pl.BlockSpec((D, D), lambda i: (0, 0)),
    ],
    out_specs=[
        pl.BlockSpec((BN, D), lambda i: (i, 0)),
        pl.BlockSpec((BN, 1), lambda i: (i, 0)),
    ],
    out_shape=[
        jax.ShapeDtypeStruct((N, D), jnp.float32),
        jax.ShapeDtypeStruct((N, 1), jnp.float32),
    ],
)


def _tc_mid_body(sp_ref, g_ref, dinv_ref, b_ref, w_ref, gout_ref):
    s = sp_ref[0] + sp_ref[1]
    dinv = dinv_ref[...]
    t = dinv * (s + g_ref[...]) + b_ref[...]
    xl = jnp.maximum(t, 0.0)
    h = jnp.dot(xl, w_ref[...], preferred_element_type=jnp.float32)
    gout_ref[...] = dinv * h


_tc_mid = pl.pallas_call(
    _tc_mid_body,
    grid=(GRID,),
    in_specs=[
        pl.BlockSpec((NC, BN, D), lambda i: (0, i, 0)),
        pl.BlockSpec((BN, D), lambda i: (i, 0)),
        pl.BlockSpec((BN, 1), lambda i: (i, 0)),
        pl.BlockSpec((1, D), lambda i: (0, 0)),
        pl.BlockSpec((D, D), lambda i: (0, 0)),
    ],
    out_specs=pl.BlockSpec((BN, D), lambda i: (i, 0)),
    out_shape=jax.ShapeDtypeStruct((N, D), jnp.float32),
)


def _tc_final_body(sp_ref, g_ref, dinv_ref, b_ref, out_ref):
    s = sp_ref[0] + sp_ref[1]
    out_ref[...] = dinv_ref[...] * (s + g_ref[...]) + b_ref[...]


_tc_final = pl.pallas_call(
    _tc_final_body,
    grid=(GRID,),
    in_specs=[
        pl.BlockSpec((NC, BN, D), lambda i: (0, i, 0)),
        pl.BlockSpec((BN, D), lambda i: (i, 0)),
        pl.BlockSpec((BN, 1), lambda i: (i, 0)),
        pl.BlockSpec((1, D), lambda i: (0, 0)),
    ],
    out_specs=pl.BlockSpec((BN, D), lambda i: (i, 0)),
    out_shape=jax.ShapeDtypeStruct((N, D), jnp.float32),
)


def kernel(x, edge_index, W1, b1, W2, b2, W3, b3):
    src = edge_index[0].astype(jnp.int32)
    dst = edge_index[1].astype(jnp.int32)
    zeros_l = jnp.zeros((ZROWS, D), jnp.float32)
    zeros_d = jnp.zeros((ROWS_PER_TILE,), jnp.float32)
    ones_d = jnp.ones((CHUNK,), jnp.float32)

    npad = (NCHUNKSP - NCHUNKS) * CHUNK
    pads = jnp.arange(npad, dtype=jnp.int32) % N
    src2 = jnp.concatenate([src, pads]).reshape(NCHUNKSP, CHUNK)
    padv = N + jnp.arange(npad, dtype=jnp.int32) % (NP - N)
    dst2 = jnp.concatenate([dst, padv]).reshape(NCHUNKSP, CHUNK)
    degp = _deg_call(dst2, ones_d, zeros_d).reshape(NC, NP, 1)
    g1, dinv = _tc_first(degp, x, W1)
    s1 = _scatter_call(g1, src2, dst2, zeros_l)
    g2 = _tc_mid(s1, g1, dinv, b1.reshape(1, D), W2)
    s2 = _scatter_call(g2, src2, dst2, zeros_l)
    g3 = _tc_mid(s2, g2, dinv, b2.reshape(1, D), W3)
    s3 = _scatter_call(g3, src2, dst2, zeros_l)
    return _tc_final(s3, g3, dinv, b3.reshape(1, D))
